# Initial kernel scaffold; baseline (speedup 1.0000x reference)
import jax
import jax.numpy as jnp
from jax.experimental import pallas as pl


def kernel(z_e, dictionary):
    raise NotImplementedError("write your pallas kernel here")


# speedup vs baseline: 4.9987x; 4.9987x over previous
import functools

import jax
import jax.numpy as jnp
from jax.experimental import pallas as pl

NUM_EMBEDDINGS = 1024
EMBEDDING_DIM = 64
SPARSITY_LEVEL = 5
COMMITMENT_COST = 0.25
EPS = 1e-10
DIVERSITY_WEIGHT = 0.001

TILE_B = 2048


def _normalize_dict(d_raw):
    norms = jnp.sqrt(jnp.sum(d_raw * d_raw, axis=0, keepdims=True))
    return d_raw / (norms + EPS)


def _omp_step_kernel(k, d_ref, res_ref, usage_prev_ref, prev_idx_ref,
                     res_out_ref, idx_out_ref, alpha_out_ref, usage_out_ref):
    t = pl.program_id(0)
    dn = _normalize_dict(d_ref[...])
    res = res_ref[...]

    corr = jax.lax.dot_general(
        dn, res, (((0,), (0,)), ((), ())),
        preferred_element_type=jnp.float32)
    v = jnp.abs(corr)

    if k > 0:
        usage = usage_prev_ref[...]
        avg = jnp.sum(usage) / NUM_EMBEDDINGS
        v = v + DIVERSITY_WEIGHT * jnp.maximum(avg - usage, 0.0)

    ii = jax.lax.broadcasted_iota(jnp.int32, (NUM_EMBEDDINGS, res.shape[1]), 0)
    for j in range(k):
        pj = prev_idx_ref[j, :]
        v = jnp.where(ii == pj[None, :], 0.0, v)

    idx = jnp.argmax(v, axis=0)
    onehot = (ii == idx[None, :]).astype(jnp.float32)

    d_sel = jax.lax.dot_general(
        dn, onehot, (((1,), (0,)), ((), ())),
        precision=jax.lax.Precision.HIGHEST,
        preferred_element_type=jnp.float32)

    num = jnp.sum(res * d_sel, axis=0, keepdims=True)
    den = jnp.sum(d_sel * d_sel, axis=0, keepdims=True)
    alpha = num / (den + EPS)

    res_out_ref[...] = res - d_sel * alpha
    idx_out_ref[...] = idx[None, :]
    alpha_out_ref[...] = alpha

    hist = jnp.sum(onehot, axis=1, keepdims=True)

    @pl.when(t == 0)
    def _init():
        if k > 0:
            usage_out_ref[...] = usage_prev_ref[...] + hist
        else:
            usage_out_ref[...] = hist

    @pl.when(t != 0)
    def _acc():
        usage_out_ref[...] += hist


def _omp_step(k, d_raw, res, usage_prev, prev_idx):
    b = res.shape[1]
    grid = (b // TILE_B,)
    in_specs = [
        pl.BlockSpec((EMBEDDING_DIM, NUM_EMBEDDINGS), lambda t: (0, 0)),
        pl.BlockSpec((EMBEDDING_DIM, TILE_B), lambda t: (0, t)),
    ]
    args = [d_raw, res]
    if k > 0:
        in_specs.append(pl.BlockSpec((NUM_EMBEDDINGS, 1), lambda t: (0, 0)))
        in_specs.append(pl.BlockSpec((k, TILE_B), lambda t: (0, t)))
        args.append(usage_prev)
        args.append(prev_idx)
        body = functools.partial(_omp_step_kernel, k)
    else:
        def body(d_ref, res_ref, *out_refs):
            _omp_step_kernel(0, d_ref, res_ref, None, None, *out_refs)

    out_shape = [
        jax.ShapeDtypeStruct((EMBEDDING_DIM, b), jnp.float32),
        jax.ShapeDtypeStruct((1, b), jnp.int32),
        jax.ShapeDtypeStruct((1, b), jnp.float32),
        jax.ShapeDtypeStruct((NUM_EMBEDDINGS, 1), jnp.float32),
    ]
    out_specs = [
        pl.BlockSpec((EMBEDDING_DIM, TILE_B), lambda t: (0, t)),
        pl.BlockSpec((1, TILE_B), lambda t: (0, t)),
        pl.BlockSpec((1, TILE_B), lambda t: (0, t)),
        pl.BlockSpec((NUM_EMBEDDINGS, 1), lambda t: (0, 0)),
    ]
    return pl.pallas_call(
        body,
        grid=grid,
        in_specs=in_specs,
        out_specs=out_specs,
        out_shape=out_shape,
    )(*args)


def _finalize_kernel(d_ref, x_ref, idx_ref, alpha_ref,
                     out_ref, coef_ref, loss_ref):
    t = pl.program_id(0)
    dn = _normalize_dict(d_ref[...])
    x = x_ref[...]

    ii = jax.lax.broadcasted_iota(jnp.int32, (NUM_EMBEDDINGS, x.shape[1]), 0)
    coef = jnp.zeros((NUM_EMBEDDINGS, x.shape[1]), jnp.float32)
    for j in range(SPARSITY_LEVEL):
        sel = ii == idx_ref[j, :][None, :]
        coef = jnp.where(sel, alpha_ref[j, :][None, :], coef)
    coef_ref[...] = coef

    z_dl = jax.lax.dot_general(
        dn, coef, (((1,), (0,)), ((), ())),
        preferred_element_type=jnp.float32)
    delta = z_dl - x
    out_ref[...] = x + delta

    part = jnp.sum(delta * delta).reshape(1, 1)

    @pl.when(t == 0)
    def _init():
        loss_ref[...] = part

    @pl.when(t != 0)
    def _acc():
        loss_ref[...] += part


def _finalize(d_raw, x, idx_hist, alpha_hist):
    b = x.shape[1]
    grid = (b // TILE_B,)
    out_shape = [
        jax.ShapeDtypeStruct((EMBEDDING_DIM, b), jnp.float32),
        jax.ShapeDtypeStruct((NUM_EMBEDDINGS, b), jnp.float32),
        jax.ShapeDtypeStruct((1, 1), jnp.float32),
    ]
    return pl.pallas_call(
        _finalize_kernel,
        grid=grid,
        in_specs=[
            pl.BlockSpec((EMBEDDING_DIM, NUM_EMBEDDINGS), lambda t: (0, 0)),
            pl.BlockSpec((EMBEDDING_DIM, TILE_B), lambda t: (0, t)),
            pl.BlockSpec((SPARSITY_LEVEL, TILE_B), lambda t: (0, t)),
            pl.BlockSpec((SPARSITY_LEVEL, TILE_B), lambda t: (0, t)),
        ],
        out_specs=[
            pl.BlockSpec((EMBEDDING_DIM, TILE_B), lambda t: (0, t)),
            pl.BlockSpec((NUM_EMBEDDINGS, TILE_B), lambda t: (0, t)),
            pl.BlockSpec((1, 1), lambda t: (0, 0)),
        ],
        out_shape=out_shape,
    )(d_raw, x, idx_hist, alpha_hist)


def kernel(z_e, dictionary):
    n, c, h, w = z_e.shape
    z = jnp.transpose(z_e, (0, 2, 3, 1))
    x = z.reshape(-1, EMBEDDING_DIM).T
    b = x.shape[1]

    res = x
    usage = None
    idx_list = []
    alpha_list = []
    for k in range(SPARSITY_LEVEL):
        prev_idx = jnp.concatenate(idx_list, axis=0) if k > 0 else None
        res, idx_k, alpha_k, usage = _omp_step(k, dictionary, res, usage,
                                               prev_idx)
        idx_list.append(idx_k)
        alpha_list.append(alpha_k)

    idx_hist = jnp.concatenate(idx_list, axis=0)
    alpha_hist = jnp.concatenate(alpha_list, axis=0)

    z_dl_st_flat, coef, loss_sum = _finalize(dictionary, x, idx_hist,
                                             alpha_hist)

    m = loss_sum[0, 0] / (n * h * w * EMBEDDING_DIM)
    loss = COMMITMENT_COST * m + m

    out1 = z_dl_st_flat.T.reshape(n, h, w, c).transpose(0, 3, 1, 2)
    return (out1, loss, coef)


# speedup vs baseline: 5.8447x; 1.1692x over previous
import functools

import jax
import jax.numpy as jnp
from jax.experimental import pallas as pl

NUM_EMBEDDINGS = 1024
EMBEDDING_DIM = 64
SPARSITY_LEVEL = 5
COMMITMENT_COST = 0.25
EPS = 1e-10
DIVERSITY_WEIGHT = 0.001

NH = 8
NL = 128

TILE_B = 2048


def _omp_step_kernel(k, d_ref, dr_ref, res_ref, usage_prev_ref, prev_idx_ref,
                     res_out_ref, idx_out_ref, alpha_out_ref, usage_out_ref):
    t = pl.program_id(0)
    d_raw = d_ref[...]
    nrm = jnp.sqrt(jnp.sum(d_raw * d_raw, axis=0, keepdims=True))
    dn = d_raw / (nrm + EPS)
    res = res_ref[...]
    bsz = res.shape[1]

    corr = jax.lax.dot_general(
        dn, res, (((0,), (0,)), ((), ())),
        preferred_element_type=jnp.float32)
    v = jnp.abs(corr).reshape(NH, NL, bsz)

    if k > 0:
        usage = usage_prev_ref[...]
        avg = jnp.sum(usage) / NUM_EMBEDDINGS
        bonus = DIVERSITY_WEIGHT * jnp.maximum(avg - usage, 0.0)
        v = v + bonus[:, :, None]

    i0 = jax.lax.broadcasted_iota(jnp.int32, (NH, NL, bsz), 0)
    i1 = jax.lax.broadcasted_iota(jnp.int32, (NH, NL, bsz), 1)
    ii = i0 * NL + i1
    for j in range(k):
        pj = prev_idx_ref[j, :]
        v = jnp.where(ii == pj[None, None, :], 0.0, v)

    l_per_slab = jnp.argmax(v, axis=1)
    m_per_slab = jnp.max(v, axis=1)
    h_star = jnp.argmax(m_per_slab, axis=0)
    ih = jax.lax.broadcasted_iota(jnp.int32, (NH, bsz), 0)
    oh_h = (ih == h_star[None, :]).astype(jnp.float32)
    l_star = jnp.sum(
        l_per_slab.astype(jnp.float32) * oh_h, axis=0).astype(jnp.int32)
    idx = h_star * NL + l_star

    il = jax.lax.broadcasted_iota(jnp.int32, (NL, bsz), 0)
    oh_l = (il == l_star[None, :]).astype(jnp.float32)

    dnr = dr_ref[...] / (nrm.reshape(NH, 1, NL) + EPS)
    u = jax.lax.dot_general(
        dnr.reshape(NH * EMBEDDING_DIM, NL), oh_l,
        (((1,), (0,)), ((), ())),
        precision=jax.lax.Precision.HIGHEST,
        preferred_element_type=jnp.float32)
    d_sel = jnp.sum(
        u.reshape(NH, EMBEDDING_DIM, bsz) * oh_h[:, None, :], axis=0)

    num = jnp.sum(res * d_sel, axis=0, keepdims=True)
    den = jnp.sum(d_sel * d_sel, axis=0, keepdims=True)
    alpha = num / (den + EPS)

    res_out_ref[...] = res - d_sel * alpha
    idx_out_ref[...] = idx[None, :]
    alpha_out_ref[...] = alpha

    hist = jax.lax.dot_general(
        oh_h, oh_l, (((1,), (1,)), ((), ())),
        precision=jax.lax.Precision.HIGHEST,
        preferred_element_type=jnp.float32)

    @pl.when(t == 0)
    def _init():
        if k > 0:
            usage_out_ref[...] = usage_prev_ref[...] + hist
        else:
            usage_out_ref[...] = hist

    @pl.when(t != 0)
    def _acc():
        usage_out_ref[...] += hist


def _omp_step(k, d_raw, d_raw_r, res, usage_prev, prev_idx):
    b = res.shape[1]
    grid = (b // TILE_B,)
    in_specs = [
        pl.BlockSpec((EMBEDDING_DIM, NUM_EMBEDDINGS), lambda t: (0, 0)),
        pl.BlockSpec((NH, EMBEDDING_DIM, NL), lambda t: (0, 0, 0)),
        pl.BlockSpec((EMBEDDING_DIM, TILE_B), lambda t: (0, t)),
    ]
    args = [d_raw, d_raw_r, res]
    if k > 0:
        in_specs.append(pl.BlockSpec((NH, NL), lambda t: (0, 0)))
        in_specs.append(pl.BlockSpec((k, TILE_B), lambda t: (0, t)))
        args.append(usage_prev)
        args.append(prev_idx)
        body = functools.partial(_omp_step_kernel, k)
    else:
        def body(d_ref, dr_ref, res_ref, *out_refs):
            _omp_step_kernel(0, d_ref, dr_ref, res_ref, None, None, *out_refs)

    out_shape = [
        jax.ShapeDtypeStruct((EMBEDDING_DIM, b), jnp.float32),
        jax.ShapeDtypeStruct((1, b), jnp.int32),
        jax.ShapeDtypeStruct((1, b), jnp.float32),
        jax.ShapeDtypeStruct((NH, NL), jnp.float32),
    ]
    out_specs = [
        pl.BlockSpec((EMBEDDING_DIM, TILE_B), lambda t: (0, t)),
        pl.BlockSpec((1, TILE_B), lambda t: (0, t)),
        pl.BlockSpec((1, TILE_B), lambda t: (0, t)),
        pl.BlockSpec((NH, NL), lambda t: (0, 0)),
    ]
    return pl.pallas_call(
        body,
        grid=grid,
        in_specs=in_specs,
        out_specs=out_specs,
        out_shape=out_shape,
    )(*args)


def _finalize_kernel(d_ref, x_ref, idx_ref, alpha_ref,
                     out_ref, coef_ref, loss_ref):
    t = pl.program_id(0)
    d_raw = d_ref[...]
    nrm = jnp.sqrt(jnp.sum(d_raw * d_raw, axis=0, keepdims=True))
    dn = d_raw / (nrm + EPS)
    x = x_ref[...]
    bsz = x.shape[1]

    ii = jax.lax.broadcasted_iota(jnp.int32, (NUM_EMBEDDINGS, bsz), 0)
    coef = jnp.zeros((NUM_EMBEDDINGS, bsz), jnp.float32)
    for j in range(SPARSITY_LEVEL):
        sel = ii == idx_ref[j, :][None, :]
        coef = jnp.where(sel, alpha_ref[j, :][None, :], coef)
    coef_ref[...] = coef

    z_dl = jax.lax.dot_general(
        dn, coef, (((1,), (0,)), ((), ())),
        preferred_element_type=jnp.float32)
    delta = z_dl - x
    out_ref[...] = x + delta

    part = jnp.sum(delta * delta).reshape(1, 1)

    @pl.when(t == 0)
    def _init():
        loss_ref[...] = part

    @pl.when(t != 0)
    def _acc():
        loss_ref[...] += part


def _finalize(d_raw, x, idx_hist, alpha_hist):
    b = x.shape[1]
    grid = (b // TILE_B,)
    out_shape = [
        jax.ShapeDtypeStruct((EMBEDDING_DIM, b), jnp.float32),
        jax.ShapeDtypeStruct((NUM_EMBEDDINGS, b), jnp.float32),
        jax.ShapeDtypeStruct((1, 1), jnp.float32),
    ]
    return pl.pallas_call(
        _finalize_kernel,
        grid=grid,
        in_specs=[
            pl.BlockSpec((EMBEDDING_DIM, NUM_EMBEDDINGS), lambda t: (0, 0)),
            pl.BlockSpec((EMBEDDING_DIM, TILE_B), lambda t: (0, t)),
            pl.BlockSpec((SPARSITY_LEVEL, TILE_B), lambda t: (0, t)),
            pl.BlockSpec((SPARSITY_LEVEL, TILE_B), lambda t: (0, t)),
        ],
        out_specs=[
            pl.BlockSpec((EMBEDDING_DIM, TILE_B), lambda t: (0, t)),
            pl.BlockSpec((NUM_EMBEDDINGS, TILE_B), lambda t: (0, t)),
            pl.BlockSpec((1, 1), lambda t: (0, 0)),
        ],
        out_shape=out_shape,
    )(d_raw, x, idx_hist, alpha_hist)


def kernel(z_e, dictionary):
    n, c, h, w = z_e.shape
    z = jnp.transpose(z_e, (0, 2, 3, 1))
    x = z.reshape(-1, EMBEDDING_DIM).T

    d_raw_r = dictionary.reshape(EMBEDDING_DIM, NH, NL).transpose(1, 0, 2)

    res = x
    usage = None
    idx_list = []
    alpha_list = []
    for k in range(SPARSITY_LEVEL):
        prev_idx = jnp.concatenate(idx_list, axis=0) if k > 0 else None
        res, idx_k, alpha_k, usage = _omp_step(k, dictionary, d_raw_r, res,
                                               usage, prev_idx)
        idx_list.append(idx_k)
        alpha_list.append(alpha_k)

    idx_hist = jnp.concatenate(idx_list, axis=0)
    alpha_hist = jnp.concatenate(alpha_list, axis=0)

    z_dl_st_flat, coef, loss_sum = _finalize(dictionary, x, idx_hist,
                                             alpha_hist)

    m = loss_sum[0, 0] / (n * h * w * EMBEDDING_DIM)
    loss = COMMITMENT_COST * m + m

    out1 = z_dl_st_flat.T.reshape(n, h, w, c).transpose(0, 3, 1, 2)
    return (out1, loss, coef)
